# baseline (speedup 1.0000x reference)
import jax
import jax.numpy as jnp
from jax.experimental import pallas as pl


def _sp_body(enc_ref, mask_ref, out_ref):
    m = mask_ref[:, 0]
    e = enc_ref[...]
    repl = jnp.float32(3.0) - jnp.float32(2.0) * m.astype(jnp.float32)
    out_ref[...] = jnp.where((m == 0)[:, None], e, repl[:, None])


def kernel(encoded, cover_img, mask):
    b, c, h, w = encoded.shape
    B = 4
    R = 512
    grid = (b // B, h // R)
    return pl.pallas_call(
        _sp_body,
        grid=grid,
        in_specs=[
            pl.BlockSpec((B, c, R, w), lambda i, j: (i, 0, j, 0)),
            pl.BlockSpec((B, 1, R, w), lambda i, j: (i, 0, j, 0)),
        ],
        out_specs=pl.BlockSpec((B, c, R, w), lambda i, j: (i, 0, j, 0)),
        out_shape=jax.ShapeDtypeStruct(encoded.shape, encoded.dtype),
    )(encoded, mask)


# speedup vs baseline: 1.0139x; 1.0139x over previous
import jax
import jax.numpy as jnp
from jax.experimental import pallas as pl
from jax.experimental.pallas import tpu as pltpu


def _sp_body(enc_ref, mask_ref, out_ref):
    m = mask_ref[:, 0]
    e = enc_ref[...]
    repl = jnp.float32(3.0) - jnp.float32(2.0) * m.astype(jnp.float32)
    out_ref[...] = jnp.where((m == 0)[:, None], e, repl[:, None])


def kernel(encoded, cover_img, mask):
    b, c, h, w = encoded.shape
    B = 2
    R = 512
    grid = (b // B, h // R)
    return pl.pallas_call(
        _sp_body,
        grid=grid,
        in_specs=[
            pl.BlockSpec((B, c, R, w), lambda i, j: (i, 0, j, 0)),
            pl.BlockSpec((B, 1, R, w), lambda i, j: (i, 0, j, 0)),
        ],
        out_specs=pl.BlockSpec((B, c, R, w), lambda i, j: (i, 0, j, 0)),
        out_shape=jax.ShapeDtypeStruct(encoded.shape, encoded.dtype),
        compiler_params=pltpu.CompilerParams(
            dimension_semantics=("parallel", "parallel"),
        ),
    )(encoded, mask)
